# baseline (speedup 1.0000x reference)
import functools

import jax
import jax.numpy as jnp
from jax.experimental import pallas as pl
from jax.experimental.pallas import tpu as pltpu

def _suppress_body(sup_ref, logits_ref, out_ref):
    x = logits_ref[...]
    col = jax.lax.broadcasted_iota(jnp.int32, x.shape, 2)
    mask = jnp.zeros(x.shape, dtype=jnp.bool_)
    for i in range(sup_ref.shape[0]):
        mask = mask | (col == sup_ref[i])
    out_ref[...] = jnp.where(mask, jnp.full_like(x, -1e9), x)


def _suppress(logits, token_ids_to_suppress):
    B, S, V = logits.shape
    grid = (B,)
    return pl.pallas_call(
        _suppress_body,
        grid_spec=pltpu.PrefetchScalarGridSpec(
            num_scalar_prefetch=1,
            grid=grid,
            in_specs=[pl.BlockSpec((1, S, V), lambda b, sup: (b, 0, 0))],
            out_specs=pl.BlockSpec((1, S, V), lambda b, sup: (b, 0, 0)),
        ),
        out_shape=jax.ShapeDtypeStruct((B, S, V), jnp.float32),
    )(token_ids_to_suppress, logits)


def kernel(logits, x, positions, attention_mask, token_ids_to_suppress, top):
    B, S, V = logits.shape
    TOP = 1000

    sup_logits = _suppress(logits, token_ids_to_suppress)

    topk_vals, topk_idx = jax.lax.top_k(sup_logits, TOP)
    log_probs = jax.nn.log_softmax(topk_vals, axis=-1)
    skey = jax.random.fold_in(jax.random.key(0), 1)
    choice = jax.random.categorical(skey, log_probs, axis=-1)
    sampled_tokens = jnp.take_along_axis(topk_idx, choice[..., None], axis=-1)[..., 0]
    chosen_logprob = jnp.take_along_axis(log_probs, choice[..., None], axis=-1)[..., 0]

    perm = jnp.argsort(positions, axis=-1)
    final_positions = jnp.take_along_axis(positions, perm, axis=-1)
    final_x = jnp.take_along_axis(x, perm, axis=-1)
    final_attention_mask = jnp.take_along_axis(attention_mask, perm, axis=-1)

    return (chosen_logprob, sampled_tokens, final_x, final_attention_mask, final_positions)
